# baseline (speedup 1.0000x reference)
import functools

import jax
import jax.numpy as jnp
from jax import lax
from jax.experimental import pallas as pl
from jax.experimental.pallas import tpu as pltpu
from jax.experimental.pallas import tpu_sc as plsc

N = 10000
E = 160000
D = 256
G = 64
H = 128
NSUB = 16
EK = 128
N2 = 10240
ROWS_PER_TILE = N2 // NSUB
EROWS_PER_TILE = 80
E2 = NSUB * EROWS_PER_TILE * EK
RB = 2000
NBLK = N // RB


def _sc_agg(zeros, hl, hr, src2, dst2):
    mesh = plsc.VectorSubcoreMesh(core_axis_name="c", subcore_axis_name="s")

    @functools.partial(
        pl.kernel,
        mesh=mesh,
        out_type=(
            jax.ShapeDtypeStruct((N2, H), jnp.float32),
            jax.ShapeDtypeStruct((N2, H), jnp.float32),
        ),
        scratch_types=[
            pltpu.VMEM((EROWS_PER_TILE, EK), jnp.int32),
            pltpu.VMEM((EROWS_PER_TILE, EK), jnp.int32),
            pltpu.VMEM((EK, H), jnp.float32),
            pltpu.VMEM_SHARED((N2, H), jnp.float32),
            pltpu.SemaphoreType.DMA,
        ],
    )
    def k(zeros_hbm, hl_hbm, hr_hbm, src_hbm, dst_hbm, aggl_hbm, aggr_hbm,
          src_v, dst_v, buf0, acc, sem0):
        c = lax.axis_index("c")
        s = lax.axis_index("s")
        rbase = s * ROWS_PER_TILE
        pltpu.sync_copy(zeros_hbm, acc.at[pl.ds(rbase, ROWS_PER_TILE)])
        pltpu.sync_copy(src_hbm.at[s], src_v)
        pltpu.sync_copy(dst_hbm.at[s], dst_v)
        plsc.subcore_barrier()

        def do(h_hbm, out_hbm):
            def chunk(j, carry):
                pltpu.async_copy(h_hbm.at[src_v.at[j]], buf0, sem0).wait()
                pltpu.sync_copy(buf0, acc.at[dst_v.at[j]], add=True)
                return carry
            lax.fori_loop(0, EROWS_PER_TILE, chunk, 0)
            plsc.subcore_barrier()
            pltpu.sync_copy(acc.at[pl.ds(rbase, ROWS_PER_TILE)],
                            out_hbm.at[pl.ds(rbase, ROWS_PER_TILE)])

        @pl.when(c == 0)
        def _():
            do(hl_hbm, aggl_hbm)

        @pl.when(c == 1)
        def _():
            do(hr_hbm, aggr_hbm)

    return k(zeros, hl, hr, src2, dst2)


def _mlp_body(eps_ref, hl_ref, hr_ref, al_ref, ar_ref, w1_ref, b1_ref,
              w2_ref, b2_ref, ol_ref, or_ref):
    h = jnp.concatenate([hl_ref[...], hr_ref[...]], axis=1)
    a = jnp.concatenate([al_ref[...], ar_ref[...]], axis=1)
    z = h * eps_ref[0, 0] + a
    z = jnp.dot(z, w1_ref[...], preferred_element_type=jnp.float32,
                precision=lax.Precision.HIGHEST) + b1_ref[...]
    z = jnp.maximum(z, 0.0)
    z = jnp.dot(z, w2_ref[...], preferred_element_type=jnp.float32,
                precision=lax.Precision.HIGHEST) + b2_ref[...]
    z = jnp.maximum(z, 0.0)
    ol_ref[...] = z[:, :H]
    or_ref[...] = z[:, H:]


def _mlp(hl, hr, al, ar, p):
    eps1 = (1.0 + p["eps"]).reshape(1, 1)
    b1 = p["b1"].reshape(1, D)
    b2 = p["b2"].reshape(1, D)
    half = pl.BlockSpec((RB, H), lambda i: (i, 0))

    def full(r, c):
        return pl.BlockSpec((r, c), lambda i: (0, 0))

    return pl.pallas_call(
        _mlp_body,
        grid=(NBLK,),
        in_specs=[
            pl.BlockSpec(memory_space=pltpu.SMEM),
            half, half, half, half,
            full(D, D), full(1, D), full(D, D), full(1, D),
        ],
        out_specs=[half, half],
        out_shape=[jax.ShapeDtypeStruct((N, H), jnp.float32)] * 2,
    )(eps1, hl, hr, al, ar, p["W1"], b1, p["W2"], b2)


def _pool_body(gid_ref, hl_ref, hr_ref, pool_ref, cnt_ref):
    @pl.when(pl.program_id(0) == 0)
    def _():
        pool_ref[...] = jnp.zeros_like(pool_ref)
        cnt_ref[...] = jnp.zeros_like(cnt_ref)

    g = gid_ref[0, 0, :]
    m = (g[:, None] == lax.broadcasted_iota(jnp.int32, (RB, G), 1))
    m = m.astype(jnp.float32)
    h = jnp.concatenate([hl_ref[...], hr_ref[...]], axis=1)
    pool_ref[...] += lax.dot_general(m, h, (((0,), (0,)), ((), ())),
                                     preferred_element_type=jnp.float32,
                                     precision=lax.Precision.HIGHEST)
    cnt_ref[...] += jnp.broadcast_to(jnp.sum(m, axis=0)[:, None], (G, D))


def _pool(hl, hr, gids):
    g3 = gids.reshape(NBLK, 1, RB)
    half = pl.BlockSpec((RB, H), lambda i: (i, 0))
    return pl.pallas_call(
        _pool_body,
        grid=(NBLK,),
        in_specs=[
            pl.BlockSpec((1, 1, RB), lambda i: (i, 0, 0)),
            half, half,
        ],
        out_specs=[
            pl.BlockSpec((G, D), lambda i: (0, 0)),
            pl.BlockSpec((G, D), lambda i: (0, 0)),
        ],
        out_shape=[jax.ShapeDtypeStruct((G, D), jnp.float32)] * 2,
    )(g3, hl, hr)


def _head_body(p1_ref, c1_ref, p2_ref, c2_ref, w1a_ref, w1b_ref, b1_ref,
               w2_ref, b2_ref, out_ref):
    m1 = p1_ref[...] / jnp.maximum(c1_ref[...], 1.0)
    m2 = p2_ref[...] / jnp.maximum(c2_ref[...], 1.0)
    z = (jnp.dot(m1, w1a_ref[...], preferred_element_type=jnp.float32,
                 precision=lax.Precision.HIGHEST)
         + jnp.dot(m2, w1b_ref[...], preferred_element_type=jnp.float32,
                   precision=lax.Precision.HIGHEST)
         + b1_ref[...])
    z = jnp.maximum(z, 0.0)
    out_ref[...] = jnp.dot(z, w2_ref[...], preferred_element_type=jnp.float32,
                           precision=lax.Precision.HIGHEST) + b2_ref[...]


def _head(p1, c1, p2, c2, dp):
    w1a = dp["W1"][:D]
    w1b = dp["W1"][D:]
    b1 = dp["b1"].reshape(1, D)
    w2 = jnp.pad(dp["W2"], ((0, 0), (0, 127)))
    b2 = jnp.pad(dp["b2"], (0, 127)).reshape(1, 128)
    out = pl.pallas_call(
        _head_body,
        out_shape=jax.ShapeDtypeStruct((G, 128), jnp.float32),
    )(p1, c1, p2, c2, w1a, w1b, b1, w2, b2)
    return out[:, :1]


def kernel(x1, edge_index1, graph_ids1, x2, edge_index2, graph_ids2,
           gin_params, dense_params):
    zeros = jnp.zeros((ROWS_PER_TILE, H), jnp.float32)
    spad = jnp.zeros((E2 - E,), jnp.int32)
    dpad = jnp.full((E2 - E,), N, jnp.int32)

    def _edges(ei):
        src = jnp.concatenate([ei[0], spad]).reshape(NSUB, EROWS_PER_TILE, EK)
        dst = jnp.concatenate([ei[1], dpad]).reshape(NSUB, EROWS_PER_TILE, EK)
        return src, dst

    src1, dst1 = _edges(edge_index1)
    src2, dst2 = _edges(edge_index2)
    h1l, h1r = x1[:, :H], x1[:, H:]
    h2l, h2r = x2[:, :H], x2[:, H:]
    for p in gin_params:
        a1l, a1r = _sc_agg(zeros, h1l, h1r, src1, dst1)
        h1l, h1r = _mlp(h1l, h1r, a1l[:N], a1r[:N], p)
        a2l, a2r = _sc_agg(zeros, h2l, h2r, src2, dst2)
        h2l, h2r = _mlp(h2l, h2r, a2l[:N], a2r[:N], p)
    p1, c1 = _pool(h1l, h1r, graph_ids1)
    p2, c2 = _pool(h2l, h2r, graph_ids2)
    return _head(p1, c1, p2, c2, dense_params)


# speedup vs baseline: 1.4015x; 1.4015x over previous
import functools

import jax
import jax.numpy as jnp
from jax import lax
from jax.experimental import pallas as pl
from jax.experimental.pallas import tpu as pltpu
from jax.experimental.pallas import tpu_sc as plsc

N = 10000
E = 160000
D = 256
G = 64
H = 128
NSUB = 16
EK = 120
N2 = 10240
ROWS_PER_TILE = N2 // NSUB
EROWS_PER_TILE = 84
E2 = NSUB * EROWS_PER_TILE * EK
RB = 2000
NBLK = N // RB


def _sc_agg(zeros, hl, hr, src2, dst2):
    mesh = plsc.VectorSubcoreMesh(core_axis_name="c", subcore_axis_name="s")

    @functools.partial(
        pl.kernel,
        mesh=mesh,
        out_type=(
            jax.ShapeDtypeStruct((N2, H), jnp.float32),
            jax.ShapeDtypeStruct((N2, H), jnp.float32),
        ),
        scratch_types=[
            pltpu.VMEM((EROWS_PER_TILE, EK), jnp.int32),
            pltpu.VMEM((EROWS_PER_TILE, EK), jnp.int32),
            pltpu.VMEM((EK, H), jnp.float32),
            pltpu.VMEM_SHARED((N2, H), jnp.float32),
            pltpu.SemaphoreType.DMA,
        ],
    )
    def k(zeros_hbm, hl_hbm, hr_hbm, src_hbm, dst_hbm, aggl_hbm, aggr_hbm,
          src_v, dst_v, buf0, acc, sem0):
        c = lax.axis_index("c")
        s = lax.axis_index("s")
        rbase = s * ROWS_PER_TILE
        pltpu.sync_copy(zeros_hbm, acc.at[pl.ds(rbase, ROWS_PER_TILE)])
        pltpu.sync_copy(src_hbm.at[s], src_v)
        pltpu.sync_copy(dst_hbm.at[s], dst_v)
        plsc.subcore_barrier()

        def do(h_hbm, out_hbm):
            def chunk(j, carry):
                pltpu.async_copy(h_hbm.at[src_v.at[j]], buf0, sem0).wait()
                pltpu.sync_copy(buf0, acc.at[dst_v.at[j]], add=True)
                return carry
            lax.fori_loop(0, EROWS_PER_TILE, chunk, 0)
            plsc.subcore_barrier()
            pltpu.sync_copy(acc.at[pl.ds(rbase, ROWS_PER_TILE)],
                            out_hbm.at[pl.ds(rbase, ROWS_PER_TILE)])

        @pl.when(c == 0)
        def _():
            do(hl_hbm, aggl_hbm)

        @pl.when(c == 1)
        def _():
            do(hr_hbm, aggr_hbm)

    return k(zeros, hl, hr, src2, dst2)


def _mlp_body(eps_ref, hl_ref, hr_ref, al_ref, ar_ref, w1_ref, b1_ref,
              w2_ref, b2_ref, ol_ref, or_ref):
    h = jnp.concatenate([hl_ref[...], hr_ref[...]], axis=1)
    a = jnp.concatenate([al_ref[...], ar_ref[...]], axis=1)
    z = h * eps_ref[0, 0] + a
    z = jnp.dot(z, w1_ref[...], preferred_element_type=jnp.float32,
                precision=lax.Precision.HIGHEST) + b1_ref[...]
    z = jnp.maximum(z, 0.0)
    z = jnp.dot(z, w2_ref[...], preferred_element_type=jnp.float32,
                precision=lax.Precision.HIGHEST) + b2_ref[...]
    z = jnp.maximum(z, 0.0)
    ol_ref[...] = z[:, :H]
    or_ref[...] = z[:, H:]


def _mlp(hl, hr, al, ar, p):
    eps1 = (1.0 + p["eps"]).reshape(1, 1)
    b1 = p["b1"].reshape(1, D)
    b2 = p["b2"].reshape(1, D)
    half = pl.BlockSpec((RB, H), lambda i: (i, 0))

    def full(r, c):
        return pl.BlockSpec((r, c), lambda i: (0, 0))

    return pl.pallas_call(
        _mlp_body,
        grid=(NBLK,),
        in_specs=[
            pl.BlockSpec(memory_space=pltpu.SMEM),
            half, half, half, half,
            full(D, D), full(1, D), full(D, D), full(1, D),
        ],
        out_specs=[half, half],
        out_shape=[jax.ShapeDtypeStruct((N, H), jnp.float32)] * 2,
    )(eps1, hl, hr, al, ar, p["W1"], b1, p["W2"], b2)


def _pool_body(gid_ref, hl_ref, hr_ref, pool_ref, cnt_ref):
    @pl.when(pl.program_id(0) == 0)
    def _():
        pool_ref[...] = jnp.zeros_like(pool_ref)
        cnt_ref[...] = jnp.zeros_like(cnt_ref)

    g = gid_ref[0, 0, :]
    m = (g[:, None] == lax.broadcasted_iota(jnp.int32, (RB, G), 1))
    m = m.astype(jnp.float32)
    h = jnp.concatenate([hl_ref[...], hr_ref[...]], axis=1)
    pool_ref[...] += lax.dot_general(m, h, (((0,), (0,)), ((), ())),
                                     preferred_element_type=jnp.float32,
                                     precision=lax.Precision.HIGHEST)
    cnt_ref[...] += jnp.broadcast_to(jnp.sum(m, axis=0)[:, None], (G, D))


def _pool(hl, hr, gids):
    g3 = gids.reshape(NBLK, 1, RB)
    half = pl.BlockSpec((RB, H), lambda i: (i, 0))
    return pl.pallas_call(
        _pool_body,
        grid=(NBLK,),
        in_specs=[
            pl.BlockSpec((1, 1, RB), lambda i: (i, 0, 0)),
            half, half,
        ],
        out_specs=[
            pl.BlockSpec((G, D), lambda i: (0, 0)),
            pl.BlockSpec((G, D), lambda i: (0, 0)),
        ],
        out_shape=[jax.ShapeDtypeStruct((G, D), jnp.float32)] * 2,
    )(g3, hl, hr)


def _head_body(p1_ref, c1_ref, p2_ref, c2_ref, w1a_ref, w1b_ref, b1_ref,
               w2_ref, b2_ref, out_ref):
    m1 = p1_ref[...] / jnp.maximum(c1_ref[...], 1.0)
    m2 = p2_ref[...] / jnp.maximum(c2_ref[...], 1.0)
    z = (jnp.dot(m1, w1a_ref[...], preferred_element_type=jnp.float32,
                 precision=lax.Precision.HIGHEST)
         + jnp.dot(m2, w1b_ref[...], preferred_element_type=jnp.float32,
                   precision=lax.Precision.HIGHEST)
         + b1_ref[...])
    z = jnp.maximum(z, 0.0)
    out_ref[...] = jnp.dot(z, w2_ref[...], preferred_element_type=jnp.float32,
                           precision=lax.Precision.HIGHEST) + b2_ref[...]


def _head(p1, c1, p2, c2, dp):
    w1a = dp["W1"][:D]
    w1b = dp["W1"][D:]
    b1 = dp["b1"].reshape(1, D)
    w2 = jnp.pad(dp["W2"], ((0, 0), (0, 127)))
    b2 = jnp.pad(dp["b2"], (0, 127)).reshape(1, 128)
    out = pl.pallas_call(
        _head_body,
        out_shape=jax.ShapeDtypeStruct((G, 128), jnp.float32),
    )(p1, c1, p2, c2, w1a, w1b, b1, w2, b2)
    return out[:, :1]


def kernel(x1, edge_index1, graph_ids1, x2, edge_index2, graph_ids2,
           gin_params, dense_params):
    zeros = jnp.zeros((ROWS_PER_TILE, H), jnp.float32)
    spad = jnp.zeros((E2 - E,), jnp.int32)
    dpad = jnp.full((E2 - E,), N, jnp.int32)

    def _edges(ei):
        src = jnp.concatenate([ei[0], spad]).reshape(NSUB, EROWS_PER_TILE, EK)
        dst = jnp.concatenate([ei[1], dpad]).reshape(NSUB, EROWS_PER_TILE, EK)
        return src, dst

    src1, dst1 = _edges(edge_index1)
    src2, dst2 = _edges(edge_index2)
    h1l, h1r = x1[:, :H], x1[:, H:]
    h2l, h2r = x2[:, :H], x2[:, H:]
    for p in gin_params:
        a1l, a1r = _sc_agg(zeros, h1l, h1r, src1, dst1)
        h1l, h1r = _mlp(h1l, h1r, a1l[:N], a1r[:N], p)
        a2l, a2r = _sc_agg(zeros, h2l, h2r, src2, dst2)
        h2l, h2r = _mlp(h2l, h2r, a2l[:N], a2r[:N], p)
    p1, c1 = _pool(h1l, h1r, graph_ids1)
    p2, c2 = _pool(h2l, h2r, graph_ids2)
    return _head(p1, c1, p2, c2, dense_params)


# speedup vs baseline: 1.6999x; 1.2129x over previous
import functools

import jax
import jax.numpy as jnp
from jax import lax
from jax.experimental import pallas as pl
from jax.experimental.pallas import tpu as pltpu
from jax.experimental.pallas import tpu_sc as plsc

N = 10000
E = 160000
D = 256
G = 64
H = 128
NSUB = 16
EK = 80
N2 = 10240
ROWS_PER_TILE = N2 // NSUB
EROWS_PER_TILE = E // (EK * NSUB)
RB = 2000
NBLK = N // RB


def _sc_agg(zeros, hl, hr, src2, dst2):
    mesh = plsc.VectorSubcoreMesh(core_axis_name="c", subcore_axis_name="s")

    @functools.partial(
        pl.kernel,
        mesh=mesh,
        out_type=(
            jax.ShapeDtypeStruct((N2, H), jnp.float32),
            jax.ShapeDtypeStruct((N2, H), jnp.float32),
        ),
        scratch_types=[
            pltpu.VMEM((EROWS_PER_TILE, EK), jnp.int32),
            pltpu.VMEM((EROWS_PER_TILE, EK), jnp.int32),
            pltpu.VMEM((EK, H), jnp.float32),
            pltpu.VMEM_SHARED((N2, H), jnp.float32),
            pltpu.SemaphoreType.DMA,
        ],
    )
    def k(zeros_hbm, hl_hbm, hr_hbm, src_hbm, dst_hbm, aggl_hbm, aggr_hbm,
          src_v, dst_v, buf0, acc, sem0):
        c = lax.axis_index("c")
        s = lax.axis_index("s")
        rbase = s * ROWS_PER_TILE
        pltpu.sync_copy(zeros_hbm, acc.at[pl.ds(rbase, ROWS_PER_TILE)])
        pltpu.sync_copy(src_hbm.at[s], src_v)
        pltpu.sync_copy(dst_hbm.at[s], dst_v)
        plsc.subcore_barrier()

        def do(h_hbm, out_hbm):
            def chunk(j, carry):
                pltpu.async_copy(h_hbm.at[src_v.at[j]], buf0, sem0).wait()
                pltpu.sync_copy(buf0, acc.at[dst_v.at[j]], add=True)
                return carry
            lax.fori_loop(0, EROWS_PER_TILE, chunk, 0)
            plsc.subcore_barrier()
            pltpu.sync_copy(acc.at[pl.ds(rbase, ROWS_PER_TILE)],
                            out_hbm.at[pl.ds(rbase, ROWS_PER_TILE)])

        @pl.when(c == 0)
        def _():
            do(hl_hbm, aggl_hbm)

        @pl.when(c == 1)
        def _():
            do(hr_hbm, aggr_hbm)

    return k(zeros, hl, hr, src2, dst2)


def _mlp_body(eps_ref, hl_ref, hr_ref, al_ref, ar_ref, w1_ref, b1_ref,
              w2_ref, b2_ref, ol_ref, or_ref):
    h = jnp.concatenate([hl_ref[...], hr_ref[...]], axis=1)
    a = jnp.concatenate([al_ref[...], ar_ref[...]], axis=1)
    z = h * eps_ref[0, 0] + a
    z = jnp.dot(z, w1_ref[...], preferred_element_type=jnp.float32,
                precision=lax.Precision.DEFAULT) + b1_ref[...]
    z = jnp.maximum(z, 0.0)
    z = jnp.dot(z, w2_ref[...], preferred_element_type=jnp.float32,
                precision=lax.Precision.DEFAULT) + b2_ref[...]
    z = jnp.maximum(z, 0.0)
    ol_ref[...] = z[:, :H]
    or_ref[...] = z[:, H:]


def _mlp(hl, hr, al, ar, p):
    eps1 = (1.0 + p["eps"]).reshape(1, 1)
    b1 = p["b1"].reshape(1, D)
    b2 = p["b2"].reshape(1, D)
    half = pl.BlockSpec((RB, H), lambda i: (i, 0))

    def full(r, c):
        return pl.BlockSpec((r, c), lambda i: (0, 0))

    return pl.pallas_call(
        _mlp_body,
        grid=(NBLK,),
        in_specs=[
            pl.BlockSpec(memory_space=pltpu.SMEM),
            half, half, half, half,
            full(D, D), full(1, D), full(D, D), full(1, D),
        ],
        out_specs=[half, half],
        out_shape=[jax.ShapeDtypeStruct((N, H), jnp.float32)] * 2,
    )(eps1, hl, hr, al, ar, p["W1"], b1, p["W2"], b2)


def _pool_body(gid_ref, hl_ref, hr_ref, pool_ref, cnt_ref):
    @pl.when(pl.program_id(0) == 0)
    def _():
        pool_ref[...] = jnp.zeros_like(pool_ref)
        cnt_ref[...] = jnp.zeros_like(cnt_ref)

    g = gid_ref[0, 0, :]
    m = (g[:, None] == lax.broadcasted_iota(jnp.int32, (RB, G), 1))
    m = m.astype(jnp.float32)
    h = jnp.concatenate([hl_ref[...], hr_ref[...]], axis=1)
    pool_ref[...] += lax.dot_general(m, h, (((0,), (0,)), ((), ())),
                                     preferred_element_type=jnp.float32,
                                     precision=lax.Precision.HIGHEST)
    cnt_ref[...] += jnp.broadcast_to(jnp.sum(m, axis=0)[:, None], (G, D))


def _pool(hl, hr, gids):
    g3 = gids.reshape(NBLK, 1, RB)
    half = pl.BlockSpec((RB, H), lambda i: (i, 0))
    return pl.pallas_call(
        _pool_body,
        grid=(NBLK,),
        in_specs=[
            pl.BlockSpec((1, 1, RB), lambda i: (i, 0, 0)),
            half, half,
        ],
        out_specs=[
            pl.BlockSpec((G, D), lambda i: (0, 0)),
            pl.BlockSpec((G, D), lambda i: (0, 0)),
        ],
        out_shape=[jax.ShapeDtypeStruct((G, D), jnp.float32)] * 2,
    )(g3, hl, hr)


def _head_body(p1_ref, c1_ref, p2_ref, c2_ref, w1a_ref, w1b_ref, b1_ref,
               w2_ref, b2_ref, out_ref):
    m1 = p1_ref[...] / jnp.maximum(c1_ref[...], 1.0)
    m2 = p2_ref[...] / jnp.maximum(c2_ref[...], 1.0)
    z = (jnp.dot(m1, w1a_ref[...], preferred_element_type=jnp.float32,
                 precision=lax.Precision.HIGHEST)
         + jnp.dot(m2, w1b_ref[...], preferred_element_type=jnp.float32,
                   precision=lax.Precision.HIGHEST)
         + b1_ref[...])
    z = jnp.maximum(z, 0.0)
    out_ref[...] = jnp.dot(z, w2_ref[...], preferred_element_type=jnp.float32,
                           precision=lax.Precision.HIGHEST) + b2_ref[...]


def _head(p1, c1, p2, c2, dp):
    w1a = dp["W1"][:D]
    w1b = dp["W1"][D:]
    b1 = dp["b1"].reshape(1, D)
    w2 = jnp.pad(dp["W2"], ((0, 0), (0, 127)))
    b2 = jnp.pad(dp["b2"], (0, 127)).reshape(1, 128)
    out = pl.pallas_call(
        _head_body,
        out_shape=jax.ShapeDtypeStruct((G, 128), jnp.float32),
    )(p1, c1, p2, c2, w1a, w1b, b1, w2, b2)
    return out[:, :1]


def kernel(x1, edge_index1, graph_ids1, x2, edge_index2, graph_ids2,
           gin_params, dense_params):
    zeros = jnp.zeros((ROWS_PER_TILE, H), jnp.float32)
    src1 = edge_index1[0].reshape(NSUB, EROWS_PER_TILE, EK)
    dst1 = edge_index1[1].reshape(NSUB, EROWS_PER_TILE, EK)
    src2 = edge_index2[0].reshape(NSUB, EROWS_PER_TILE, EK)
    dst2 = edge_index2[1].reshape(NSUB, EROWS_PER_TILE, EK)
    h1l, h1r = x1[:, :H], x1[:, H:]
    h2l, h2r = x2[:, :H], x2[:, H:]
    for p in gin_params:
        a1l, a1r = _sc_agg(zeros, h1l, h1r, src1, dst1)
        h1l, h1r = _mlp(h1l, h1r, a1l, a1r, p)
        a2l, a2r = _sc_agg(zeros, h2l, h2r, src2, dst2)
        h2l, h2r = _mlp(h2l, h2r, a2l, a2r, p)
    p1, c1 = _pool(h1l, h1r, graph_ids1)
    p2, c2 = _pool(h2l, h2r, graph_ids2)
    return _head(p1, c1, p2, c2, dense_params)
